# Initial kernel scaffold; baseline (speedup 1.0000x reference)
import jax
import jax.numpy as jnp
from jax.experimental import pallas as pl


def kernel(x_actuators, x_sensors):
    raise NotImplementedError("write your pallas kernel here")


# speedup vs baseline: 5.8863x; 5.8863x over previous
import functools

import jax
import jax.numpy as jnp
from jax import lax
from jax.experimental import pallas as pl

K = 16


def _topk_kernel(act_ref, sens_ref, vals_ref, idxs_ref):
    act = act_ref[0]
    sens = sens_ref[0]
    A, L = act.shape
    S = sens.shape[0]

    xn = jnp.sqrt(jnp.sum(act * act, axis=1, keepdims=True))
    yn = jnp.sqrt(jnp.sum(sens * sens, axis=1, keepdims=True))

    num = lax.dot_general(
        act, sens,
        dimension_numbers=(((1,), (1,)), ((), ())),
        precision=lax.Precision.DEFAULT,
        preferred_element_type=jnp.float32,
    )
    sim = num / (xn * yn.reshape(1, S))

    score = sim * sim
    iota = lax.broadcasted_iota(jnp.int32, (A, S), 1)
    for j in range(K):
        m = jnp.max(score, axis=1, keepdims=True)
        is_max = score >= m
        idx = jnp.min(jnp.where(is_max, iota, S), axis=1, keepdims=True)
        sel = iota == idx
        val = jnp.sum(jnp.where(sel, sim, 0.0), axis=1)
        vals_ref[0, :, j] = val
        idxs_ref[0, :, j] = idx[:, 0]
        score = jnp.where(sel, -1.0, score)


@jax.jit
def kernel(x_actuators, x_sensors):
    B, A, L = x_actuators.shape
    S = x_sensors.shape[1]
    k = K

    vals, idxs = pl.pallas_call(
        _topk_kernel,
        grid=(B,),
        in_specs=[
            pl.BlockSpec((1, A, L), lambda b: (b, 0, 0)),
            pl.BlockSpec((1, S, L), lambda b: (b, 0, 0)),
        ],
        out_specs=[
            pl.BlockSpec((1, A, k), lambda b: (b, 0, 0)),
            pl.BlockSpec((1, A, k), lambda b: (b, 0, 0)),
        ],
        out_shape=[
            jax.ShapeDtypeStruct((B, A, k), jnp.float32),
            jax.ShapeDtypeStruct((B, A, k), jnp.int32),
        ],
    )(x_actuators, x_sensors)

    target_nodes = idxs.reshape(B, A * k)
    source_nodes = jnp.tile(jnp.repeat(jnp.arange(A), k)[None, :], (B, 1))
    edges = jnp.stack([source_nodes, target_nodes], axis=1)
    weights = vals.reshape(B, A * k)
    return edges, weights


# speedup vs baseline: 6.0753x; 1.0321x over previous
import jax
import jax.numpy as jnp
from jax import lax
from jax.experimental import pallas as pl

K = 16


def _topk_kernel(act_ref, sens_ref, vals_ref, idxs_ref):
    act = act_ref[0]
    sens = sens_ref[0]
    A, L = act.shape
    S = sens.shape[0]

    xn = jnp.sqrt(jnp.sum(act * act, axis=1))
    yn = jnp.sqrt(jnp.sum(sens * sens, axis=1))

    num_t = lax.dot_general(
        act, sens,
        dimension_numbers=(((1,), (1,)), ((), ())),
        precision=lax.Precision.DEFAULT,
        preferred_element_type=jnp.float32,
    ).T
    sim = num_t / (yn[:, None] * xn[None, :])

    score = sim * sim
    iota = lax.broadcasted_iota(jnp.int32, (S, A), 0)
    packed = iota * 2 + jnp.where(sim < 0.0, 1, 0)
    big = jnp.int32(1 << 30)
    for j in range(K):
        m = jnp.max(score, axis=0)
        cand = jnp.where(score >= m[None, :], packed, big)
        idxp = jnp.min(cand, axis=0)
        r = jnp.sqrt(m)
        val = jnp.where((idxp & 1) == 1, -r, r)
        vals_ref[0, j, :] = val
        idxs_ref[0, j, :] = lax.shift_right_logical(idxp, 1)
        score = jnp.where(packed == idxp[None, :], -1.0, score)


@jax.jit
def kernel(x_actuators, x_sensors):
    B, A, L = x_actuators.shape
    S = x_sensors.shape[1]
    k = K

    vals_t, idxs_t = pl.pallas_call(
        _topk_kernel,
        grid=(B,),
        in_specs=[
            pl.BlockSpec((1, A, L), lambda b: (b, 0, 0)),
            pl.BlockSpec((1, S, L), lambda b: (b, 0, 0)),
        ],
        out_specs=[
            pl.BlockSpec((1, k, A), lambda b: (b, 0, 0)),
            pl.BlockSpec((1, k, A), lambda b: (b, 0, 0)),
        ],
        out_shape=[
            jax.ShapeDtypeStruct((B, k, A), jnp.float32),
            jax.ShapeDtypeStruct((B, k, A), jnp.int32),
        ],
    )(x_actuators, x_sensors)

    target_nodes = jnp.swapaxes(idxs_t, 1, 2).reshape(B, A * k)
    source_nodes = jnp.tile(jnp.repeat(jnp.arange(A), k)[None, :], (B, 1))
    edges = jnp.stack([source_nodes, target_nodes], axis=1)
    weights = jnp.swapaxes(vals_t, 1, 2).reshape(B, A * k)
    return edges, weights


# speedup vs baseline: 6.0821x; 1.0011x over previous
import jax
import jax.numpy as jnp
from jax import lax
from jax.experimental import pallas as pl
from jax.experimental.pallas import tpu as pltpu

K = 16


def _topk_kernel(act_ref, sens_ref, vals_ref, idxs_ref):
    act = act_ref[0]
    sens = sens_ref[0]
    A, L = act.shape
    S = sens.shape[0]

    xn = jnp.sqrt(jnp.sum(act * act, axis=1))
    yn = jnp.sqrt(jnp.sum(sens * sens, axis=1))

    num_t = lax.dot_general(
        act, sens,
        dimension_numbers=(((1,), (1,)), ((), ())),
        precision=lax.Precision.DEFAULT,
        preferred_element_type=jnp.float32,
    ).T
    sim = num_t / (yn[:, None] * xn[None, :])

    score = sim * sim
    iota = lax.broadcasted_iota(jnp.int32, (S, A), 0)
    packed = iota * 2 + jnp.where(sim < 0.0, 1, 0)
    big = jnp.int32(1 << 30)
    for j in range(K):
        m = jnp.max(score, axis=0)
        cand = jnp.where(score >= m[None, :], packed, big)
        idxp = jnp.min(cand, axis=0)
        r = jnp.sqrt(m)
        val = jnp.where((idxp & 1) == 1, -r, r)
        vals_ref[0, j, :] = val
        idxs_ref[0, j, :] = lax.shift_right_logical(idxp, 1)
        score = jnp.where(packed == idxp[None, :], -1.0, score)


@jax.jit
def kernel(x_actuators, x_sensors):
    B, A, L = x_actuators.shape
    S = x_sensors.shape[1]
    k = K

    vals_t, idxs_t = pl.pallas_call(
        _topk_kernel,
        grid=(B,),
        compiler_params=pltpu.CompilerParams(
            dimension_semantics=("parallel",),
        ),
        in_specs=[
            pl.BlockSpec((1, A, L), lambda b: (b, 0, 0)),
            pl.BlockSpec((1, S, L), lambda b: (b, 0, 0)),
        ],
        out_specs=[
            pl.BlockSpec((1, k, A), lambda b: (b, 0, 0)),
            pl.BlockSpec((1, k, A), lambda b: (b, 0, 0)),
        ],
        out_shape=[
            jax.ShapeDtypeStruct((B, k, A), jnp.float32),
            jax.ShapeDtypeStruct((B, k, A), jnp.int32),
        ],
    )(x_actuators, x_sensors)

    target_nodes = jnp.swapaxes(idxs_t, 1, 2).reshape(B, A * k)
    source_nodes = jnp.tile(jnp.repeat(jnp.arange(A), k)[None, :], (B, 1))
    edges = jnp.stack([source_nodes, target_nodes], axis=1)
    weights = jnp.swapaxes(vals_t, 1, 2).reshape(B, A * k)
    return edges, weights
